# baseline (speedup 1.0000x reference)
import functools

import jax
import jax.numpy as jnp
from jax import lax
from jax.experimental import pallas as pl
from jax.experimental.pallas import tpu as pltpu
from jax.experimental.pallas import tpu_sc as plsc

HID = 768
EPS = 1e-12


def _sc_gather_ln(idx_flat, table, ptt, lnp, rows_per_pos):
    n = idx_flat.shape[0]
    info = plsc.get_sparse_core_info()
    nw = info.num_cores * info.num_subcores
    per_w = n // nw
    chunk = 32
    n_chunk = per_w // chunk
    assert per_w % chunk == 0 and rows_per_pos % chunk == 0

    mesh = plsc.VectorSubcoreMesh(core_axis_name="c", subcore_axis_name="s")

    def _ln_16rows(rows_ref, b, g, ptt_ref, lnp_ref, s_idx):
        ridx = lax.iota(jnp.int32, 16) + g * 16
        acc_s = jnp.zeros((16,), jnp.float32)
        acc_q = jnp.zeros((16,), jnp.float32)

        def pass1(co, carry):
            a_s, a_q = carry
            pv = ptt_ref[s_idx, pl.ds(co * 16, 16)]
            for cc in range(16):
                c = co * 16 + cc
                cidx = jnp.full((16,), c, jnp.int32)
                x = plsc.load_gather(rows_ref.at[b], [ridx, cidx])
                x = x + pv[cc]
                a_s = a_s + x
                a_q = a_q + x * x
            return (a_s, a_q)

        acc_s, acc_q = lax.fori_loop(0, HID // 16, pass1, (acc_s, acc_q))
        mu = acc_s * (1.0 / HID)
        var = acc_q * (1.0 / HID) - mu * mu
        t = var + EPS
        i = plsc.bitcast(t, jnp.int32)
        i = jnp.int32(0x5F3759DF) - lax.shift_right_logical(i, 1)
        y = plsc.bitcast(i, jnp.float32)
        for _ in range(3):
            y = y * (1.5 - 0.5 * t * y * y)

        def pass2(co, carry):
            pv = ptt_ref[s_idx, pl.ds(co * 16, 16)]
            wv = lnp_ref[0, pl.ds(co * 16, 16)]
            bv = lnp_ref[1, pl.ds(co * 16, 16)]
            for cc in range(16):
                c = co * 16 + cc
                cidx = jnp.full((16,), c, jnp.int32)
                x = plsc.load_gather(rows_ref.at[b], [ridx, cidx])
                x = x + pv[cc]
                out = (x - mu) * y * wv[cc] + bv[cc]
                plsc.store_scatter(rows_ref.at[b], [ridx, cidx], out)
            return carry

        lax.fori_loop(0, HID // 16, pass2, 0)

    @functools.partial(
        pl.kernel,
        out_type=jax.ShapeDtypeStruct((n, HID), jnp.float32),
        mesh=mesh,
        compiler_params=pltpu.CompilerParams(use_tc_tiling_on_sc=True,
                                             needs_layout_passes=False),
        scratch_types=[
            pltpu.VMEM((per_w,), jnp.int32),
            pltpu.VMEM((2, chunk, HID), jnp.float32),
            pltpu.VMEM(ptt.shape, jnp.float32),
            pltpu.VMEM((2, HID), jnp.float32),
            pltpu.SemaphoreType.DMA,
            pltpu.SemaphoreType.DMA,
            pltpu.SemaphoreType.DMA,
        ],
    )
    def gather_kernel(idx_hbm, table_hbm, ptt_hbm, lnp_hbm, out_hbm,
                      idx_v, rows_v, ptt_v, lnp_v, isem, g0, g1):
        wid = lax.axis_index("s") * info.num_cores + lax.axis_index("c")
        base = wid * per_w
        pltpu.async_copy(idx_hbm.at[pl.ds(base, per_w)], idx_v, isem).wait()
        pltpu.async_copy(ptt_hbm, ptt_v, isem).wait()
        pltpu.async_copy(lnp_hbm, lnp_v, isem).wait()

        def pair(j, carry):
            k0 = j * 2
            o0 = k0 * chunk
            h0 = pltpu.async_copy(
                table_hbm.at[idx_v.at[pl.ds(o0, chunk)]], rows_v.at[0], g0)
            h1 = pltpu.async_copy(
                table_hbm.at[idx_v.at[pl.ds(o0 + chunk, chunk)]],
                rows_v.at[1], g1)
            h0.wait()
            s0 = (base + o0) // rows_per_pos
            _ln_16rows(rows_v, 0, 0, ptt_v, lnp_v, s0)
            _ln_16rows(rows_v, 0, 1, ptt_v, lnp_v, s0)
            pltpu.sync_copy(rows_v.at[0], out_hbm.at[pl.ds(base + o0, chunk)])
            h1.wait()
            s1 = (base + o0 + chunk) // rows_per_pos
            _ln_16rows(rows_v, 1, 0, ptt_v, lnp_v, s1)
            _ln_16rows(rows_v, 1, 1, ptt_v, lnp_v, s1)
            pltpu.sync_copy(rows_v.at[1],
                            out_hbm.at[pl.ds(base + o0 + chunk, chunk)])
            return carry

        lax.fori_loop(0, n_chunk // 2, pair, 0)

    return gather_kernel(idx_flat, table, ptt, lnp)


def _ln_rows(x, w, b):
    mu = jnp.mean(x, axis=-1, keepdims=True)
    d = x - mu
    var = jnp.mean(d * d, axis=-1, keepdims=True)
    return d * lax.rsqrt(var + EPS) * w + b


def _image_side(feat_t, loc_t2, img_W, loc_W, iparams, bb=1024):
    nbox, b, vfeat = feat_t.shape
    jb = b // bb

    def body(f_ref, l_ref, w_ref, lw_ref, p_ref, o_ref):
        acc = lax.dot_general(f_ref[...].reshape(bb, vfeat), w_ref[...],
                              (((1,), (0,)), ((), ())),
                              preferred_element_type=jnp.float32)
        img = _ln_rows(acc + p_ref[0:1], p_ref[1:2], p_ref[2:3])
        lacc = lax.dot_general(l_ref[...], lw_ref[...],
                               (((0,), (0,)), ((), ())),
                               preferred_element_type=jnp.float32)
        loc_e = _ln_rows(lacc + p_ref[3:4], p_ref[4:5], p_ref[5:6])
        y = _ln_rows(img + loc_e + p_ref[8:9], p_ref[6:7], p_ref[7:8])
        o_ref[...] = y.reshape(1, bb, HID)

    return pl.pallas_call(
        body,
        grid=(nbox, jb),
        in_specs=[
            pl.BlockSpec((1, bb, vfeat), lambda i, j: (i, j, 0)),
            pl.BlockSpec((5, bb), lambda i, j, _jb=jb: (0, i * _jb + j)),
            pl.BlockSpec((vfeat, HID), lambda i, j: (0, 0)),
            pl.BlockSpec((5, HID), lambda i, j: (0, 0)),
            pl.BlockSpec((9, HID), lambda i, j: (0, 0)),
        ],
        out_specs=pl.BlockSpec((1, bb, HID), lambda i, j: (i, j, 0)),
        out_shape=jax.ShapeDtypeStruct((nbox, b, HID), jnp.float32),
    )(feat_t, loc_t2, img_W, loc_W, iparams)


def kernel(token_ids, image_feat, image_loc, word_emb, pos_emb, type_emb,
           ln_w, ln_b, img_W, img_b, loc_W, loc_b,
           img_ln_w, img_ln_b, loc_ln_w, loc_ln_b, v_ln_w, v_ln_b):
    b, s = token_ids.shape
    nbox, vfeat = image_feat.shape[1], image_feat.shape[2]

    idx_t = token_ids.T.reshape(-1).astype(jnp.int32)
    ptt = pos_emb[:s] + type_emb[0]
    lnp = jnp.stack([ln_w, ln_b])
    text_flat = _sc_gather_ln(idx_t, word_emb, ptt, lnp, rows_per_pos=b)
    text_t = text_flat.reshape(s, b, HID)

    feat_t = image_feat.transpose(1, 0, 2)
    loc_t2 = image_loc.transpose(2, 1, 0).reshape(image_loc.shape[2], -1)
    iparams = jnp.stack([img_b, img_ln_w, img_ln_b,
                         loc_b, loc_ln_w, loc_ln_b,
                         v_ln_w, v_ln_b, type_emb[1]])
    v_t = _image_side(feat_t, loc_t2, img_W, loc_W, iparams)

    return (text_t.transpose(1, 0, 2), v_t.transpose(1, 0, 2))


# speedup vs baseline: 6.0742x; 6.0742x over previous
import functools

import jax
import jax.numpy as jnp
from jax import lax
from jax.experimental import pallas as pl
from jax.experimental.pallas import tpu as pltpu
from jax.experimental.pallas import tpu_sc as plsc

HID = 768
EPS = 1e-12


def _sc_gather(idx_flat, table):
    n = idx_flat.shape[0]
    info = plsc.get_sparse_core_info()
    nw = info.num_cores * info.num_subcores
    per_w = n // nw
    chunk = 80
    n_pair = per_w // (2 * chunk)
    assert per_w % (2 * chunk) == 0 and chunk % 8 == 0

    mesh = plsc.VectorSubcoreMesh(core_axis_name="c", subcore_axis_name="s")

    @functools.partial(
        pl.kernel,
        out_type=jax.ShapeDtypeStruct((n, HID), jnp.float32),
        mesh=mesh,
        compiler_params=pltpu.CompilerParams(use_tc_tiling_on_sc=True),
        scratch_types=[
            pltpu.VMEM((per_w,), jnp.int32),
            pltpu.VMEM((2, chunk, HID), jnp.float32),
            pltpu.SemaphoreType.DMA,
            pltpu.SemaphoreType.DMA,
            pltpu.SemaphoreType.DMA,
        ],
    )
    def gather_kernel(idx_hbm, table_hbm, out_hbm, idx_v, rows_v, isem, g0, g1):
        wid = lax.axis_index("s") * info.num_cores + lax.axis_index("c")
        base = wid * per_w
        pltpu.async_copy(idx_hbm.at[pl.ds(base, per_w)], idx_v, isem).wait()

        def body(j, carry):
            o0 = j * (2 * chunk)
            h0 = pltpu.async_copy(
                table_hbm.at[idx_v.at[pl.ds(o0, chunk)]], rows_v.at[0], g0)
            h1 = pltpu.async_copy(
                table_hbm.at[idx_v.at[pl.ds(o0 + chunk, chunk)]], rows_v.at[1], g1)
            h0.wait()
            pltpu.sync_copy(rows_v.at[0], out_hbm.at[pl.ds(base + o0, chunk)])
            h1.wait()
            pltpu.sync_copy(rows_v.at[1],
                            out_hbm.at[pl.ds(base + o0 + chunk, chunk)])
            return carry

        lax.fori_loop(0, n_pair, body, 0)

    return gather_kernel(idx_flat, table)


def _ln_rows(x, w, b):
    mu = jnp.mean(x, axis=-1, keepdims=True)
    d = x - mu
    var = jnp.mean(d * d, axis=-1, keepdims=True)
    return d * lax.rsqrt(var + EPS) * w + b


def _text_ln(gathered_t, ptt, lnp, b, s):
    def body(g_ref, ptt_ref, lnp_ref, o_ref):
        x = g_ref[...] + ptt_ref[0]
        y = _ln_rows(x, lnp_ref[0:1], lnp_ref[1:2])
        o_ref[...] = y.reshape(1, b, HID)

    return pl.pallas_call(
        body,
        grid=(s,),
        in_specs=[
            pl.BlockSpec((b, HID), lambda i: (i, 0)),
            pl.BlockSpec((1, 1, HID), lambda i: (i, 0, 0)),
            pl.BlockSpec((2, HID), lambda i: (0, 0)),
        ],
        out_specs=pl.BlockSpec((1, b, HID), lambda i: (i, 0, 0)),
        out_shape=jax.ShapeDtypeStruct((s, b, HID), jnp.float32),
    )(gathered_t, ptt.reshape(s, 1, HID), lnp)


def _image_side(feat_t, loc_t2, img_W, loc_W, iparams, bb=1024):
    nbox, b, vfeat = feat_t.shape
    jb = b // bb

    def body(f_ref, l_ref, w_ref, lw_ref, p_ref, o_ref):
        acc = lax.dot_general(f_ref[...].reshape(bb, vfeat), w_ref[...],
                              (((1,), (0,)), ((), ())),
                              preferred_element_type=jnp.float32)
        img = _ln_rows(acc + p_ref[0:1], p_ref[1:2], p_ref[2:3])
        lacc = lax.dot_general(l_ref[...], lw_ref[...],
                               (((0,), (0,)), ((), ())),
                               preferred_element_type=jnp.float32)
        loc_e = _ln_rows(lacc + p_ref[3:4], p_ref[4:5], p_ref[5:6])
        y = _ln_rows(img + loc_e + p_ref[8:9], p_ref[6:7], p_ref[7:8])
        o_ref[...] = y.reshape(1, bb, HID)

    return pl.pallas_call(
        body,
        grid=(nbox, jb),
        in_specs=[
            pl.BlockSpec((1, bb, vfeat), lambda i, j: (i, j, 0)),
            pl.BlockSpec((5, bb), lambda i, j, _jb=jb: (0, i * _jb + j)),
            pl.BlockSpec((vfeat, HID), lambda i, j: (0, 0)),
            pl.BlockSpec((5, HID), lambda i, j: (0, 0)),
            pl.BlockSpec((9, HID), lambda i, j: (0, 0)),
        ],
        out_specs=pl.BlockSpec((1, bb, HID), lambda i, j: (i, j, 0)),
        out_shape=jax.ShapeDtypeStruct((nbox, b, HID), jnp.float32),
    )(feat_t, loc_t2, img_W, loc_W, iparams)


def kernel(token_ids, image_feat, image_loc, word_emb, pos_emb, type_emb,
           ln_w, ln_b, img_W, img_b, loc_W, loc_b,
           img_ln_w, img_ln_b, loc_ln_w, loc_ln_b, v_ln_w, v_ln_b):
    b, s = token_ids.shape
    nbox, vfeat = image_feat.shape[1], image_feat.shape[2]

    idx_t = token_ids.T.reshape(-1).astype(jnp.int32)
    gathered_t = _sc_gather(idx_t, word_emb)

    ptt = pos_emb[:s] + type_emb[0]
    lnp = jnp.stack([ln_w, ln_b])
    text_t = _text_ln(gathered_t, ptt, lnp, b, s)

    feat_t = image_feat.transpose(1, 0, 2)
    loc_t2 = image_loc.transpose(2, 1, 0).reshape(image_loc.shape[2], -1)
    iparams = jnp.stack([img_b, img_ln_w, img_ln_b,
                         loc_b, loc_ln_w, loc_ln_b,
                         v_ln_w, v_ln_b, type_emb[1]])
    v_t = _image_side(feat_t, loc_t2, img_W, loc_W, iparams)

    return (text_t.transpose(1, 0, 2), v_t.transpose(1, 0, 2))


# speedup vs baseline: 6.1038x; 1.0049x over previous
import functools

import jax
import jax.numpy as jnp
from jax import lax
from jax.experimental import pallas as pl
from jax.experimental.pallas import tpu as pltpu
from jax.experimental.pallas import tpu_sc as plsc

HID = 768
EPS = 1e-12


def _sc_gather(idx_flat, table):
    n = idx_flat.shape[0]
    info = plsc.get_sparse_core_info()
    nw = info.num_cores * info.num_subcores
    per_w = n // nw
    chunk = 80
    n_pair = per_w // (2 * chunk)
    assert per_w % (2 * chunk) == 0 and chunk % 8 == 0

    mesh = plsc.VectorSubcoreMesh(core_axis_name="c", subcore_axis_name="s")

    @functools.partial(
        pl.kernel,
        out_type=jax.ShapeDtypeStruct((n, HID), jnp.float32),
        mesh=mesh,
        compiler_params=pltpu.CompilerParams(use_tc_tiling_on_sc=True),
        scratch_types=[
            pltpu.VMEM((per_w,), jnp.int32),
            pltpu.VMEM((2, chunk, HID), jnp.float32),
            pltpu.SemaphoreType.DMA,
            pltpu.SemaphoreType.DMA,
            pltpu.SemaphoreType.DMA,
        ],
    )
    def gather_kernel(idx_hbm, table_hbm, out_hbm, idx_v, rows_v, isem, g0, g1):
        wid = lax.axis_index("s") * info.num_cores + lax.axis_index("c")
        base = wid * per_w
        pltpu.async_copy(idx_hbm.at[pl.ds(base, per_w)], idx_v, isem).wait()

        def body(j, carry):
            o0 = j * (2 * chunk)
            h0 = pltpu.async_copy(
                table_hbm.at[idx_v.at[pl.ds(o0, chunk)]], rows_v.at[0], g0)
            h1 = pltpu.async_copy(
                table_hbm.at[idx_v.at[pl.ds(o0 + chunk, chunk)]], rows_v.at[1], g1)
            h0.wait()
            pltpu.sync_copy(rows_v.at[0], out_hbm.at[pl.ds(base + o0, chunk)])
            h1.wait()
            pltpu.sync_copy(rows_v.at[1],
                            out_hbm.at[pl.ds(base + o0 + chunk, chunk)])
            return carry

        lax.fori_loop(0, n_pair, body, 0)

    return gather_kernel(idx_flat, table)


def _ln_rows(x, w, b):
    mu = jnp.mean(x, axis=-1, keepdims=True)
    d = x - mu
    var = jnp.mean(d * d, axis=-1, keepdims=True)
    return d * lax.rsqrt(var + EPS) * w + b


def _text_ln(gathered_t, ptt, lnp, b, s, tp=2):
    def body(g_ref, ptt_ref, lnp_ref, o_ref):
        x = g_ref[...].reshape(tp, b, HID) + ptt_ref[...]
        o_ref[...] = _ln_rows(x, lnp_ref[0:1], lnp_ref[1:2])

    return pl.pallas_call(
        body,
        grid=(s // tp,),
        in_specs=[
            pl.BlockSpec((tp * b, HID), lambda i: (i, 0)),
            pl.BlockSpec((tp, 1, HID), lambda i: (i, 0, 0)),
            pl.BlockSpec((2, HID), lambda i: (0, 0)),
        ],
        out_specs=pl.BlockSpec((tp, b, HID), lambda i: (i, 0, 0)),
        out_shape=jax.ShapeDtypeStruct((s, b, HID), jnp.float32),
    )(gathered_t, ptt.reshape(s, 1, HID), lnp)


def _image_side(feat_t, loc_t2, img_W, loc_W, iparams, bb=1024):
    nbox, b, vfeat = feat_t.shape
    jb = b // bb

    def body(f_ref, l_ref, w_ref, lw_ref, p_ref, o_ref):
        acc = lax.dot_general(f_ref[...].reshape(bb, vfeat), w_ref[...],
                              (((1,), (0,)), ((), ())),
                              preferred_element_type=jnp.float32)
        img = _ln_rows(acc + p_ref[0:1], p_ref[1:2], p_ref[2:3])
        lacc = lax.dot_general(l_ref[...], lw_ref[...],
                               (((0,), (0,)), ((), ())),
                               preferred_element_type=jnp.float32)
        loc_e = _ln_rows(lacc + p_ref[3:4], p_ref[4:5], p_ref[5:6])
        y = _ln_rows(img + loc_e + p_ref[8:9], p_ref[6:7], p_ref[7:8])
        o_ref[...] = y.reshape(1, bb, HID)

    return pl.pallas_call(
        body,
        grid=(nbox, jb),
        in_specs=[
            pl.BlockSpec((1, bb, vfeat), lambda i, j: (i, j, 0)),
            pl.BlockSpec((5, bb), lambda i, j, _jb=jb: (0, i * _jb + j)),
            pl.BlockSpec((vfeat, HID), lambda i, j: (0, 0)),
            pl.BlockSpec((5, HID), lambda i, j: (0, 0)),
            pl.BlockSpec((9, HID), lambda i, j: (0, 0)),
        ],
        out_specs=pl.BlockSpec((1, bb, HID), lambda i, j: (i, j, 0)),
        out_shape=jax.ShapeDtypeStruct((nbox, b, HID), jnp.float32),
    )(feat_t, loc_t2, img_W, loc_W, iparams)


def kernel(token_ids, image_feat, image_loc, word_emb, pos_emb, type_emb,
           ln_w, ln_b, img_W, img_b, loc_W, loc_b,
           img_ln_w, img_ln_b, loc_ln_w, loc_ln_b, v_ln_w, v_ln_b):
    b, s = token_ids.shape
    nbox, vfeat = image_feat.shape[1], image_feat.shape[2]

    idx_t = token_ids.T.reshape(-1).astype(jnp.int32)
    gathered_t = _sc_gather(idx_t, word_emb)

    ptt = pos_emb[:s] + type_emb[0]
    lnp = jnp.stack([ln_w, ln_b])
    text_t = _text_ln(gathered_t, ptt, lnp, b, s)

    feat_t = image_feat.transpose(1, 0, 2)
    loc_t2 = image_loc.transpose(2, 1, 0).reshape(image_loc.shape[2], -1)
    iparams = jnp.stack([img_b, img_ln_w, img_ln_b,
                         loc_b, loc_ln_w, loc_ln_b,
                         v_ln_w, v_ln_b, type_emb[1]])
    v_t = _image_side(feat_t, loc_t2, img_W, loc_W, iparams)

    return (text_t.transpose(1, 0, 2), v_t.transpose(1, 0, 2))
